# Initial kernel scaffold; baseline (speedup 1.0000x reference)
import jax
import jax.numpy as jnp
from jax.experimental import pallas as pl


def kernel(x, output_size, kernel_size, dilation, padding, stride):
    raise NotImplementedError("write your pallas kernel here")


# speedup vs baseline: 4.1181x; 4.1181x over previous
import functools

import jax
import jax.numpy as jnp
from jax import lax
from jax.experimental import pallas as pl
from jax.experimental.pallas import tpu as pltpu
from jax.experimental.pallas import tpu_sc as plsc

H = 224
C = 96
R = 16
NSTRIP = H // R
WPAD = 256
NCORES = 2
NSUB = 16
NW = NCORES * NSUB
CPW = C // NW
NCHUNK = H // 16


def _build_sc_call():
    mesh = plsc.VectorSubcoreMesh(core_axis_name="c", subcore_axis_name="s")

    @functools.partial(
        pl.kernel,
        out_type=jax.ShapeDtypeStruct((C, H, H), jnp.float32),
        mesh=mesh,
        compiler_params=pltpu.CompilerParams(use_tc_tiling_on_sc=False),
        scratch_types=[
            pltpu.VMEM((9, R, WPAD), jnp.float32),
            pltpu.VMEM((R, H), jnp.float32),
        ],
    )
    def col2im_sc(x_hbm, out_hbm, ibuf, obuf):
        wid = lax.axis_index("s") * NCORES + lax.axis_index("c")
        zeros16 = jnp.zeros((16,), jnp.float32)

        def zrow(r, carry):
            for j in range(9):
                ibuf[j, r, pl.ds(0, 16)] = zeros16
                ibuf[j, r, pl.ds(240, 16)] = zeros16
            return carry

        lax.fori_loop(0, R, zrow, 0)

        for ci in range(CPW):
            c = wid * CPW + ci

            def strip(s, carry):
                h0 = s * R
                for kh in range(3):
                    for kw in range(3):
                        j = kh * 3 + kw
                        if kh == 1:
                            pltpu.sync_copy(
                                x_hbm.at[c, 1, kw, pl.ds(h0, R), :],
                                ibuf.at[j, :, pl.ds(16, H)])
                        elif kh == 0:
                            @pl.when(s < NSTRIP - 1)
                            def _():
                                pltpu.sync_copy(
                                    x_hbm.at[c, 0, kw, pl.ds(h0 + 1, R), :],
                                    ibuf.at[j, :, pl.ds(16, H)])

                            @pl.when(s == NSTRIP - 1)
                            def _():
                                pltpu.sync_copy(
                                    x_hbm.at[c, 0, kw, pl.ds(h0 + 1, R - 1), :],
                                    ibuf.at[j, pl.ds(0, R - 1), pl.ds(16, H)])
                                for ch in range(NCHUNK):
                                    ibuf[j, R - 1, pl.ds(16 + ch * 16, 16)] = zeros16
                        else:
                            @pl.when(s > 0)
                            def _():
                                pltpu.sync_copy(
                                    x_hbm.at[c, 2, kw, pl.ds(h0 - 1, R), :],
                                    ibuf.at[j, :, pl.ds(16, H)])

                            @pl.when(s == 0)
                            def _():
                                pltpu.sync_copy(
                                    x_hbm.at[c, 2, kw, pl.ds(0, R - 1), :],
                                    ibuf.at[j, pl.ds(1, R - 1), pl.ds(16, H)])
                                for ch in range(NCHUNK):
                                    ibuf[j, 0, pl.ds(16 + ch * 16, 16)] = zeros16

                def row(r, cc):
                    for ch in range(NCHUNK):
                        b = ch * 16
                        acc = ibuf[0, r, pl.ds(b + 17, 16)]
                        acc = acc + ibuf[1, r, pl.ds(b + 16, 16)]
                        acc = acc + ibuf[2, r, pl.ds(b + 15, 16)]
                        acc = acc + ibuf[3, r, pl.ds(b + 17, 16)]
                        acc = acc + ibuf[4, r, pl.ds(b + 16, 16)]
                        acc = acc + ibuf[5, r, pl.ds(b + 15, 16)]
                        acc = acc + ibuf[6, r, pl.ds(b + 17, 16)]
                        acc = acc + ibuf[7, r, pl.ds(b + 16, 16)]
                        acc = acc + ibuf[8, r, pl.ds(b + 15, 16)]
                        obuf[r, pl.ds(b, 16)] = acc
                    return cc

                lax.fori_loop(0, R, row, 0)
                pltpu.sync_copy(obuf, out_hbm.at[c, pl.ds(h0, R), :])
                return carry

            lax.fori_loop(0, NSTRIP, strip, 0)

    return col2im_sc


_COL2IM_SC = _build_sc_call()


def kernel(x, output_size, kernel_size, dilation, padding, stride):
    x4 = x.reshape(C, 3, 3, H, H)
    out = _COL2IM_SC(x4)
    return out.reshape(1, C, H, H)


# speedup vs baseline: 7.0144x; 1.7033x over previous
import functools

import jax
import jax.numpy as jnp
from jax import lax
from jax.experimental import pallas as pl
from jax.experimental.pallas import tpu as pltpu
from jax.experimental.pallas import tpu_sc as plsc

H = 224
C = 96
R = 16
NSTRIP = H // R
WPAD = 256
NCORES = 2
NSUB = 16
NW = NCORES * NSUB
CPW = C // NW
TPW = CPW * NSTRIP
NCHUNK = H // 16


def _build_sc_call():
    mesh = plsc.VectorSubcoreMesh(core_axis_name="c", subcore_axis_name="s")

    @functools.partial(
        pl.kernel,
        out_type=jax.ShapeDtypeStruct((C, H, H), jnp.float32),
        mesh=mesh,
        compiler_params=pltpu.CompilerParams(use_tc_tiling_on_sc=False),
        scratch_types=[
            pltpu.VMEM((2, 9, R, WPAD), jnp.float32),
            pltpu.VMEM((2, R, H), jnp.float32),
            pltpu.SemaphoreType.DMA,
            pltpu.SemaphoreType.DMA,
            pltpu.SemaphoreType.DMA,
            pltpu.SemaphoreType.DMA,
        ],
    )
    def col2im_sc(x_hbm, out_hbm, ibuf, obuf, isem0, isem1, osem0, osem1):
        wid = lax.axis_index("s") * NCORES + lax.axis_index("c")
        base_t = wid * TPW
        isem = (isem0, isem1)
        osem = (osem0, osem1)
        zeros16 = jnp.zeros((16,), jnp.float32)

        def zrow(r, carry):
            for b in range(2):
                for p in range(9):
                    ibuf[b, p, r, pl.ds(0, 16)] = zeros16
                    ibuf[b, p, r, pl.ds(240, 16)] = zeros16
            return carry

        lax.fori_loop(0, R, zrow, 0)

        def split(t):
            c = t // NSTRIP
            s = t - c * NSTRIP
            return c, s

        def in_copies(t, b):
            c, s = split(t)
            h0 = s * R
            cps = []
            for p in range(9):
                kh = p // 3
                row0 = p * H + h0 + 1 - kh
                cps.append(pltpu.make_async_copy(
                    x_hbm.at[c, pl.ds(row0, R), :],
                    ibuf.at[b, p, :, pl.ds(16, H)],
                    isem[b]))
            return cps

        def out_copy(t, b):
            c, s = split(t)
            return pltpu.make_async_copy(
                obuf.at[b], out_hbm.at[c, pl.ds(s * R, R), :], osem[b])

        def issue_in(t, b):
            for cp in in_copies(t, b):
                cp.start()

        def wait_in(t, b):
            for cp in in_copies(t, b):
                cp.wait()

        def compute(t, b):
            _, s = split(t)

            @pl.when(s == 0)
            def _():
                for p in (6, 7, 8):
                    for ch in range(NCHUNK):
                        ibuf[b, p, 0, pl.ds(16 + ch * 16, 16)] = zeros16

            @pl.when(s == NSTRIP - 1)
            def _():
                for p in (0, 1, 2):
                    for ch in range(NCHUNK):
                        ibuf[b, p, R - 1, pl.ds(16 + ch * 16, 16)] = zeros16

            def rowf(r, cc):
                for ch in range(NCHUNK):
                    bc = ch * 16
                    acc = ibuf[b, 0, r, pl.ds(bc + 17, 16)]
                    acc = acc + ibuf[b, 1, r, pl.ds(bc + 16, 16)]
                    acc = acc + ibuf[b, 2, r, pl.ds(bc + 15, 16)]
                    acc = acc + ibuf[b, 3, r, pl.ds(bc + 17, 16)]
                    acc = acc + ibuf[b, 4, r, pl.ds(bc + 16, 16)]
                    acc = acc + ibuf[b, 5, r, pl.ds(bc + 15, 16)]
                    acc = acc + ibuf[b, 6, r, pl.ds(bc + 17, 16)]
                    acc = acc + ibuf[b, 7, r, pl.ds(bc + 16, 16)]
                    acc = acc + ibuf[b, 8, r, pl.ds(bc + 15, 16)]
                    obuf[b, r, pl.ds(bc, 16)] = acc
                return cc

            lax.fori_loop(0, R, rowf, 0)

        issue_in(base_t, 0)

        def pair(g, carry):
            t0 = base_t + 2 * g
            issue_in(t0 + 1, 1)
            wait_in(t0, 0)

            @pl.when(g > 0)
            def _():
                out_copy(t0, 0).wait()

            compute(t0, 0)
            out_copy(t0, 0).start()

            @pl.when(g < TPW // 2 - 1)
            def _():
                issue_in(t0 + 2, 0)

            wait_in(t0 + 1, 1)

            @pl.when(g > 0)
            def _():
                out_copy(t0 + 1, 1).wait()

            compute(t0 + 1, 1)
            out_copy(t0 + 1, 1).start()
            return carry

        lax.fori_loop(0, TPW // 2, pair, 0)
        out_copy(base_t, 0).wait()
        out_copy(base_t, 1).wait()

    return col2im_sc


_COL2IM_SC = _build_sc_call()


def kernel(x, output_size, kernel_size, dilation, padding, stride):
    x3 = x.reshape(C, 9 * H, H)
    out = _COL2IM_SC(x3)
    return out.reshape(1, C, H, H)


# speedup vs baseline: 8.5714x; 1.2220x over previous
import functools

import jax
import jax.numpy as jnp
from jax import lax
from jax.experimental import pallas as pl
from jax.experimental.pallas import tpu as pltpu
from jax.experimental.pallas import tpu_sc as plsc

H = 224
C = 96
R = 16
NSTRIP = H // R
NCORES = 2
NSUB = 16
NW = NCORES * NSUB
CPW = C // NW
TPW = CPW * NSTRIP
NCHUNK = H // 16
PLANE = H * H
STRIPW = R * H
PAD = 240
BUFW = PAD + STRIPW + PAD
STMAX = (H - R) * H


def _build_sc_call():
    mesh = plsc.VectorSubcoreMesh(core_axis_name="c", subcore_axis_name="s")

    @functools.partial(
        pl.kernel,
        out_type=jax.ShapeDtypeStruct((C, H, H), jnp.float32),
        mesh=mesh,
        compiler_params=pltpu.CompilerParams(use_tc_tiling_on_sc=False),
        scratch_types=[
            pltpu.VMEM((2, 9, BUFW), jnp.float32),
            pltpu.VMEM((2, R, H), jnp.float32),
            pltpu.SemaphoreType.DMA,
            pltpu.SemaphoreType.DMA,
            pltpu.SemaphoreType.DMA,
            pltpu.SemaphoreType.DMA,
        ],
    )
    def col2im_sc(x_hbm, out_hbm, ibuf, obuf, isem0, isem1, osem0, osem1):
        wid = lax.axis_index("s") * NCORES + lax.axis_index("c")
        base_t = wid * TPW
        isem = (isem0, isem1)
        osem = (osem0, osem1)
        zeros16 = jnp.zeros((16,), jnp.float32)
        lane_f = lax.iota(jnp.int32, 16).astype(jnp.float32)
        mask_lo = jnp.minimum(lane_f, 1.0)
        mask_hi = jnp.minimum(15.0 - lane_f, 1.0)

        def zpad(i, carry):
            for b in range(2):
                for p in range(9):
                    ibuf[b, p, pl.ds(i * 16, 16)] = zeros16
                    ibuf[b, p, pl.ds(PAD + STRIPW + i * 16, 16)] = zeros16
            return carry

        lax.fori_loop(0, PAD // 16, zpad, 0)

        def split(t):
            c = t // NSTRIP
            s = t - c * NSTRIP
            return c, s

        def starts(t):
            _, s = split(t)
            h0 = s * R
            st = []
            for kh in range(3):
                raw = (h0 + 1 - kh) * H
                st.append(pl.multiple_of(jnp.clip(raw, 0, STMAX), H))
            return st

        def in_copies(t, b):
            c, _ = split(t)
            st = starts(t)
            cps = []
            for p in range(9):
                kh = p // 3
                cps.append(pltpu.make_async_copy(
                    x_hbm.at[c * 9 + p, pl.ds(st[kh], STRIPW)],
                    ibuf.at[b, p, pl.ds(PAD, STRIPW)],
                    isem[b]))
            return cps

        def out_copy(t, b):
            c, s = split(t)
            return pltpu.make_async_copy(
                obuf.at[b], out_hbm.at[c, pl.ds(s * R, R), :], osem[b])

        def issue_in(t, b):
            for cp in in_copies(t, b):
                cp.start()

        def wait_in(t, b):
            for cp in in_copies(t, b):
                cp.wait()

        def compute(t, b):
            _, s = split(t)
            h0 = s * R
            st = starts(t)
            rb = [PAD + (h0 + 1 - kh) * H - st[kh] for kh in range(3)]

            def rowf(r, cc):
                rowbase = r * H
                b0 = rb[0] + rowbase
                b1 = rb[1] + rowbase
                b2 = rb[2] + rowbase
                for ch in range(NCHUNK):
                    o0 = ch * 16 + 1
                    o1 = ch * 16
                    o2 = ch * 16 - 1
                    t0 = ibuf[b, 0, pl.ds(b0 + o0, 16)]
                    t3 = ibuf[b, 3, pl.ds(b1 + o0, 16)]
                    t6 = ibuf[b, 6, pl.ds(b2 + o0, 16)]
                    acc0 = t0 + t3 + t6
                    if ch == NCHUNK - 1:
                        acc0 = acc0 * mask_hi
                    t1 = ibuf[b, 1, pl.ds(b0 + o1, 16)]
                    t4 = ibuf[b, 4, pl.ds(b1 + o1, 16)]
                    t7 = ibuf[b, 7, pl.ds(b2 + o1, 16)]
                    acc1 = t1 + t4 + t7
                    t2 = ibuf[b, 2, pl.ds(b0 + o2, 16)]
                    t5 = ibuf[b, 5, pl.ds(b1 + o2, 16)]
                    t8 = ibuf[b, 8, pl.ds(b2 + o2, 16)]
                    acc2 = t2 + t5 + t8
                    if ch == 0:
                        acc2 = acc2 * mask_lo
                    obuf[b, r, pl.ds(ch * 16, 16)] = acc0 + acc1 + acc2
                return cc

            lax.fori_loop(0, R, rowf, 0)

        issue_in(base_t, 0)

        def pair(g, carry):
            t0 = base_t + 2 * g
            issue_in(t0 + 1, 1)
            wait_in(t0, 0)

            @pl.when(g > 0)
            def _():
                out_copy(t0, 0).wait()

            compute(t0, 0)
            out_copy(t0, 0).start()

            @pl.when(g < TPW // 2 - 1)
            def _():
                issue_in(t0 + 2, 0)

            wait_in(t0 + 1, 1)

            @pl.when(g > 0)
            def _():
                out_copy(t0 + 1, 1).wait()

            compute(t0 + 1, 1)
            out_copy(t0 + 1, 1).start()
            return carry

        lax.fori_loop(0, TPW // 2, pair, 0)
        out_copy(base_t, 0).wait()
        out_copy(base_t, 1).wait()

    return col2im_sc


_COL2IM_SC = _build_sc_call()


def kernel(x, output_size, kernel_size, dilation, padding, stride):
    x2 = x.reshape(C * 9, H * H)
    out = _COL2IM_SC(x2)
    return out.reshape(1, C, H, H)
